# Initial kernel scaffold; baseline (speedup 1.0000x reference)
import jax
import jax.numpy as jnp
from jax.experimental import pallas as pl


def kernel(x, params, targets):
    raise NotImplementedError("write your pallas kernel here")


# speedup vs baseline: 1.0499x; 1.0499x over previous
import functools
import math

import jax
import jax.numpy as jnp
import numpy as np
from jax.experimental import pallas as pl
from jax.experimental.pallas import tpu as pltpu

E = 4
B = 1024
C = 256
BT = 32
NBLK = 40
B_PAD = NBLK * BT
NITER = 15


def _conv2d(x, w, b):
    y = jax.lax.conv_general_dilated(x, w, window_strides=(1, 1), padding='SAME',
                                     dimension_numbers=('NCHW', 'OIHW', 'NCHW'))
    return y + b[None, :, None, None]


def _avgpool(x, k):
    return jax.lax.reduce_window(x, 0.0, jax.lax.add, (1, 1, k, k), (1, 1, k, k), 'VALID') / float(k * k)


def _scorer_trunk(p, e, x):
    h = jax.nn.relu(_conv2d(x, p[f'e{e}_sc_w'], p[f'e{e}_sc_b']))
    k = h.shape[2] // 4
    h = _avgpool(h, k)
    h = h.reshape(h.shape[0], -1)
    h = jax.nn.relu(h @ p[f'e{e}_sfc_w'] + p[f'e{e}_sfc_b'])
    return h


def _scores_noisy(x, params, targets):
    Bn = x.shape[0]
    feats = [_scorer_trunk(params, e, x) for e in range(E)]
    scores = jnp.stack([(feats[e] @ params[f'e{e}_sh_w'] + params[f'e{e}_sh_b'])[:, 0]
                        for e in range(E)], axis=1)
    class_logits = jnp.stack([feats[e] @ params[f'e{e}_scl_w'] + params[f'e{e}_scl_b']
                              for e in range(E)], axis=1)
    class_probs = jax.nn.softmax(class_logits, axis=2)
    tgt = jnp.broadcast_to(targets[:, None].astype(jnp.int32), (Bn, E))
    gt_probs = jnp.take_along_axis(class_probs, tgt[:, :, None], axis=2)[:, :, 0]
    combined = scores + 1.0 * jnp.log(jnp.clip(gt_probs, 1e-9, None))
    return combined / 0.1


def _waterfall_body(sn_ref, assign_ref, slot_ref, perm_ref, bexp_ref):
    sn = sn_ref[:]
    f32 = jnp.float32
    i32 = jnp.int32

    def iota_f32(shape, dim):
        return jax.lax.broadcasted_iota(i32, shape, dim).astype(f32)

    rj = jax.lax.broadcasted_iota(i32, (B, B), 0)
    ci = jax.lax.broadcasted_iota(i32, (B, B), 1)
    ltt = (rj <= ci).astype(f32)

    iota_e = iota_f32((E, B), 0)
    iota_e1 = iota_f32((E, 1), 0)

    assign = jnp.zeros((E, B), f32)
    for it in range(NITER):
        cap = jnp.sum(assign, axis=1, keepdims=True)
        rem = 1.0 - jnp.sum(assign, axis=0, keepdims=True)
        deficit = jnp.clip(cap * (1.0 / C), 0.0, 1.0)
        s = sn - deficit
        s = jnp.where(cap >= C, -1e30, s)
        m = jnp.max(s, axis=0, keepdims=True)
        cand = jnp.where(s == m, iota_e, float(E))
        sel = jnp.min(cand, axis=0, keepdims=True)
        onehot = (iota_e == sel).astype(f32)
        want = onehot * rem
        rank = jnp.dot(want, ltt, preferred_element_type=f32)
        space = jnp.minimum(C - cap, float(2 ** it))
        take = want * (rank <= space).astype(f32)
        assign = assign + take

    cap = jnp.sum(assign, axis=1, keepdims=True)
    rem = 1.0 - jnp.sum(assign, axis=0, keepdims=True)
    mn = jnp.min(cap, axis=0, keepdims=True)
    cand = jnp.where(cap == mn, iota_e1, float(E))
    least = jnp.min(cand, axis=0, keepdims=True)
    assign = assign + (iota_e1 == least).astype(f32) * rem
    assign_ref[:] = assign

    count = jnp.sum(assign, axis=1, keepdims=True)
    pc = jnp.floor((count + (BT - 1)) * (1.0 / BT)) * BT
    slt4 = (jax.lax.broadcasted_iota(i32, (E, E), 0) >
            jax.lax.broadcasted_iota(i32, (E, E), 1)).astype(f32)
    starts = jnp.dot(slt4, pc, preferred_element_type=f32)
    rank_all = jnp.dot(assign, ltt, preferred_element_type=f32)
    slotv = jnp.sum(assign * (starts + rank_all - 1.0), axis=0, keepdims=True)
    slot_ref[:] = slotv.astype(jnp.int32)

    oh = (iota_f32((B_PAD, B), 0) == slotv).astype(f32)
    idx_col = iota_f32((B, 1), 0)
    perm = jnp.dot(oh, idx_col, preferred_element_type=f32)
    perm_ref[:] = perm.astype(jnp.int32)

    bstart = starts * (1.0 / BT)
    iota_g = iota_f32((E, NBLK), 1)
    bexp = jnp.sum((iota_g >= bstart).astype(f32), axis=0, keepdims=True) - 1.0
    bexp_ref[:] = bexp.astype(jnp.int32)


@jax.jit
def _waterfall_route(sn_t):
    return pl.pallas_call(
        _waterfall_body,
        out_shape=(
            jax.ShapeDtypeStruct((E, B), jnp.float32),
            jax.ShapeDtypeStruct((1, B), jnp.int32),
            jax.ShapeDtypeStruct((B_PAD, 1), jnp.int32),
            jax.ShapeDtypeStruct((1, NBLK), jnp.int32),
        ),
    )(sn_t)


def _bn(x, g, b):
    return g[None, :, None, None] * x / np.sqrt(1.0 + 1e-5) + b[None, :, None, None]


def _maxpool2(x):
    return jax.lax.reduce_window(x, -jnp.inf, jax.lax.max, (1, 1, 2, 2), (1, 1, 2, 2), 'VALID')


def _encoder(p, e, x):
    h = jax.nn.relu(_bn(_conv2d(x, p[f'e{e}_c1_w'], p[f'e{e}_c1_b']), p[f'e{e}_bn1_g'], p[f'e{e}_bn1_b']))
    h = jax.nn.relu(_bn(_conv2d(h, p[f'e{e}_c2_w'], p[f'e{e}_c2_b']), p[f'e{e}_bn2_g'], p[f'e{e}_bn2_b']))
    h = _maxpool2(h)
    h = jax.nn.relu(_bn(_conv2d(h, p[f'e{e}_c3_w'], p[f'e{e}_c3_b']), p[f'e{e}_bn3_g'], p[f'e{e}_bn3_b']))
    h = _maxpool2(h)
    h = jax.nn.relu(_bn(_conv2d(h, p[f'e{e}_c4_w'], p[f'e{e}_c4_b']), p[f'e{e}_bn4_g'], p[f'e{e}_bn4_b']))
    h = jnp.mean(h, axis=(2, 3))
    return h


def kernel(x, params, targets):
    sn = _scores_noisy(x, params, targets)
    assign, slot, perm, bexp = _waterfall_route(sn.T)
    out = jnp.zeros((B, 10), jnp.float32)
    for e in range(E):
        fe = _encoder(params, e, x)
        z = fe @ params[f'e{e}_pr_w'] + params[f'e{e}_pr_b']
        logits = z @ params[f'e{e}_cl_w'] + params[f'e{e}_cl_b']
        out = jnp.where(assign[e][:, None] > 0.5, logits, out)
    return out
